# Initial kernel scaffold; baseline (speedup 1.0000x reference)
import jax
import jax.numpy as jnp
from jax.experimental import pallas as pl


def kernel(query_pos, support_pos, q_idx, s_idx, support_normals, support_curvature, W1, b1, W2, b2):
    raise NotImplementedError("write your pallas kernel here")


# speedup vs baseline: 2.4318x; 2.4318x over previous
import functools

import jax
import jax.numpy as jnp
from jax import lax
from jax.experimental import pallas as pl
from jax.experimental.pallas import tpu as pltpu
from jax.experimental.pallas import tpu_sc as plsc

NC = 2
NS = 16
NW = NC * NS

GCH = 128
GNB = 2

SCH = 2048
SCHW = SCH // 2
FB = 128
VBUF = 160
RPT_MAX = 800


def _gelu_exact(z):
    return 0.5 * z * (1.0 + lax.erf(z * 0.7071067811865476))


def _round_up(x, m):
    return (x + m - 1) // m * m


def _partials_kernel(sf_ref, qp_ref, w1s_ref, w1q_ref, b1_ref, a_ref, b_ref):
    a_ref[...] = jnp.dot(sf_ref[...], w1s_ref[...],
                         preferred_element_type=jnp.float32) + b1_ref[...]
    b_ref[...] = jnp.dot(qp_ref[...], w1q_ref[...],
                         preferred_element_type=jnp.float32)


def _gather_kernel(a_hbm, b_hbm, sidx_hbm, qidx_hbm, z1_hbm,
                   sv0, sv1, qv0, qv1, bs0, bs1, bb0, bb1,
                   semi, sema, semb, semst, *, epw):
    wid = lax.axis_index("s") * NC + lax.axis_index("c")
    base = wid * epw
    svs = [sv0, sv1]
    qvs = [qv0, qv1]
    bss = [bs0, bs1]
    bbs = [bb0, bb1]

    def sc_body(g, _):
        sbase = base + g * (GNB * GCH)
        cps = []
        for b in range(GNB):
            cb = sbase + b * GCH
            cps.append(pltpu.async_copy(sidx_hbm.at[pl.ds(cb, GCH)], svs[b], semi))
            cps.append(pltpu.async_copy(qidx_hbm.at[pl.ds(cb, GCH)], qvs[b], semi))
        for c in cps:
            c.wait()
        cps = [pltpu.async_copy(a_hbm.at[svs[b]], bss[b], sema)
               for b in range(GNB)]
        cps += [pltpu.async_copy(b_hbm.at[qvs[b]], bbs[b], semb)
                for b in range(GNB)]
        for c in cps:
            c.wait()
        for b in range(GNB):
            bsf = bss[b]
            bbf = bbs[b]

            def add_body(i, _, bsf=bsf, bbf=bbf):
                r = bsf.at[i]
                rb = bbf.at[i]
                for f in range(8):
                    sl = pl.ds(f * 16, 16)
                    r[sl] = r[sl] + rb[sl]
                return 0
            lax.fori_loop(0, GCH, add_body, 0)
        cps = [pltpu.async_copy(
                   bss[b], z1_hbm.at[pl.ds(sbase + b * GCH, GCH)], semst)
               for b in range(GNB)]
        for c in cps:
            c.wait()
        return 0

    lax.fori_loop(0, epw // (GNB * GCH), sc_body, 0)


def _mlp_kernel(z1_ref, w2_ref, b2_ref, h2_ref):
    h1 = _gelu_exact(z1_ref[...])
    z2 = jnp.dot(h1, w2_ref[...], preferred_element_type=jnp.float32) + b2_ref[...]
    h2_ref[...] = _gelu_exact(z2)


def _scatter_kernel(qidx_hbm, h2_hbm, out_hbm,
                    qbuf, qbufb, valbuf, eidbuf, rbuf, hbuf, acc,
                    sem, semqa, semqb,
                    *, epad, strips, rpt):
    cid = lax.axis_index("c")
    sid = lax.axis_index("s")
    wid = sid * NC + cid
    nchunks = epad // SCH
    lanes = lax.iota(jnp.int32, 16)

    for s in range(strips):
        lo = (s * NW + wid) * rpt

        def zero_body(i, _):
            acc[pl.ds(i * 16, 16)] = jnp.zeros((16,), jnp.float32)
            return 0
        lax.fori_loop(0, rpt * 128 // 16, zero_body, 0)

        def flush(cnt, ngroups):
            for g in range(FB // 16):
                v = valbuf[pl.ds(g * 16, 16)]
                eidbuf[pl.ds(g * 16, 16)] = v >> 10
                rbuf[pl.ds(g * 16, 16)] = v & 1023
            pltpu.async_copy(h2_hbm.at[eidbuf], hbuf, sem).wait()

            def acc_group(g, _):
                rvec = rbuf[pl.ds(g * 16, 16)]
                for k in range(16):
                    r = rvec[k]
                    hb = hbuf.at[g * 16 + k]
                    for f in range(8):
                        sl = pl.ds(r * 128 + f * 16, 16)
                        acc[sl] = jnp.maximum(acc[sl], hb[pl.ds(f * 16, 16)])
                return 0
            lax.fori_loop(0, ngroups, acc_group, 0)
            valbuf[pl.ds(0, 16)] = valbuf[pl.ds(FB, 16)]
            return cnt - FB

        def scan_chunk(buf, ci, cnt):
            def half(cnt, mask, rel, eidv):
                pc = plsc.all_reduce_population_count(mask)[0]

                def compact(cnt):
                    packed = (eidv << 10) | jnp.where(mask, rel, 0)
                    key = jnp.where(mask, 0, 1)
                    _, vals = plsc.sort_key_val(key, packed)
                    valbuf[pl.ds(cnt, 16)] = vals
                    cnt = cnt + pc
                    return lax.cond(cnt >= FB, lambda c: flush(c, FB // 16),
                                    lambda c: c, cnt)
                return lax.cond(pc > 0, compact, lambda c: c, cnt)

            def group_body(j, cnt):
                w = buf[pl.ds(j * 16, 16)]
                qa = jnp.bitwise_and(w, 65535)
                qb = lax.shift_right_logical(w, 16)
                rela = qa - lo
                relb = qb - lo
                maska = (rela >= 0) & (rela < rpt)
                maskb = (relb >= 0) & (relb < rpt)
                pcany = plsc.all_reduce_population_count(maska | maskb)[0]

                def both(cnt):
                    wi = ci * SCHW + j * 16 + lanes
                    cnt = half(cnt, maska, rela, wi * 2)
                    return half(cnt, maskb, relb, wi * 2 + 1)
                return lax.cond(pcany > 0, both, lambda c: c, cnt)

            return lax.fori_loop(0, SCHW // 16, group_body, cnt)

        def fire(ci, buf, sem):
            pltpu.async_copy(qidx_hbm.at[pl.ds(ci * SCHW, SCHW)], buf, sem)

        def wait(ci, buf, sem):
            pltpu.make_async_copy(
                qidx_hbm.at[pl.ds(ci * SCHW, SCHW)], buf, sem).wait()

        npairs = nchunks // 2
        fire(0, qbuf, semqa)

        def pair_body(p, cnt):
            c0 = 2 * p
            fire(c0 + 1, qbufb, semqb)
            wait(c0, qbuf, semqa)
            cnt = scan_chunk(qbuf, c0, cnt)

            @pl.when(p < npairs - 1)
            def _():
                fire(c0 + 2, qbuf, semqa)
            wait(c0 + 1, qbufb, semqb)
            return scan_chunk(qbufb, c0 + 1, cnt)

        cnt = lax.fori_loop(0, npairs, pair_body, jnp.int32(0))

        def final_flush(c):
            valbuf[pl.ds(c, 16)] = jnp.full((16,), rpt, jnp.int32)
            return flush(c, (c + 15) // 16)
        lax.cond(cnt > 0, final_flush, lambda c: c, cnt)
        pltpu.sync_copy(acc.at[pl.ds(0, rpt * 128)],
                        out_hbm.at[pl.ds(lo * 128, rpt * 128)])


@jax.jit
def kernel(query_pos, support_pos, q_idx, s_idx, support_normals,
           support_curvature, W1, b1, W2, b2):
    n = query_pos.shape[0]
    e = q_idx.shape[0]
    h = W1.shape[1]

    epad = _round_up(e, NW * 1024)
    strips = -(-n // (NW * RPT_MAX))
    rpt = -(-n // (strips * NW))
    npad = strips * NW * rpt
    epw = epad // NW

    nb = 1000
    npadp = _round_up(n, nb)
    padn = npadp - n
    sfeat = jnp.concatenate(
        [support_pos, support_normals, support_curvature,
         jnp.zeros((n, 1), jnp.float32)], axis=1)
    sfeat = jnp.concatenate(
        [sfeat, jnp.zeros((padn, 8), jnp.float32)], axis=0)
    qp = jnp.concatenate(
        [query_pos, jnp.zeros((n, 1), jnp.float32)], axis=1)
    qp = jnp.concatenate([qp, jnp.zeros((padn, 4), jnp.float32)], axis=0)
    w1s = jnp.concatenate([W1, jnp.zeros((1, h), jnp.float32)], axis=0)
    w1q = jnp.concatenate([-W1[0:3], jnp.zeros((1, h), jnp.float32)], axis=0)

    a_arr, b_arr = pl.pallas_call(
        _partials_kernel,
        grid=(npadp // nb,),
        in_specs=[
            pl.BlockSpec((nb, 8), lambda i: (i, 0)),
            pl.BlockSpec((nb, 4), lambda i: (i, 0)),
            pl.BlockSpec((8, h), lambda i: (0, 0)),
            pl.BlockSpec((4, h), lambda i: (0, 0)),
            pl.BlockSpec((1, h), lambda i: (0, 0)),
        ],
        out_specs=[pl.BlockSpec((nb, h), lambda i: (i, 0)),
                   pl.BlockSpec((nb, h), lambda i: (i, 0))],
        out_shape=[jax.ShapeDtypeStruct((npadp, h), jnp.float32),
                   jax.ShapeDtypeStruct((npadp, h), jnp.float32)],
    )(sfeat, qp, w1s, w1q, b1.reshape(1, h))

    pad = epad - e
    s_pad = jnp.concatenate([s_idx, jnp.zeros((pad,), jnp.int32)])
    q_pad_g = jnp.concatenate([q_idx, jnp.zeros((pad,), jnp.int32)])
    q_pad_s = jnp.concatenate([q_idx, jnp.full((pad,), 65535, jnp.int32)])
    q2c = q_pad_s.reshape(-1, 2)
    q_pack = jnp.bitwise_or(q2c[:, 0], q2c[:, 1] << 16)

    mesh = plsc.VectorSubcoreMesh(core_axis_name="c", subcore_axis_name="s")

    z1 = pl.kernel(
        functools.partial(_gather_kernel, epw=epw),
        out_type=jax.ShapeDtypeStruct((epad, h), jnp.float32),
        mesh=mesh,
        compiler_params=pltpu.CompilerParams(needs_layout_passes=False),
        scratch_types=(
            [pltpu.VMEM((GCH,), jnp.int32)] * 4
            + [pltpu.VMEM((GCH, h), jnp.float32)] * 4
            + [pltpu.SemaphoreType.DMA] * 4
        ),
    )(a_arr, b_arr, s_pad, q_pad_g)

    be = 2048
    h2 = pl.pallas_call(
        _mlp_kernel,
        grid=(epad // be,),
        in_specs=[
            pl.BlockSpec((be, h), lambda i: (i, 0)),
            pl.BlockSpec((h, h), lambda i: (0, 0)),
            pl.BlockSpec((1, h), lambda i: (0, 0)),
        ],
        out_specs=pl.BlockSpec((be, h), lambda i: (i, 0)),
        out_shape=jax.ShapeDtypeStruct((epad, h), jnp.float32),
    )(z1, W2, b2.reshape(1, h))

    out1d = pl.kernel(
        functools.partial(_scatter_kernel, epad=epad, strips=strips, rpt=rpt),
        out_type=jax.ShapeDtypeStruct((npad * 128,), jnp.float32),
        mesh=mesh,
        compiler_params=pltpu.CompilerParams(needs_layout_passes=False),
        scratch_types=[
            pltpu.VMEM((SCHW,), jnp.int32),
            pltpu.VMEM((SCHW,), jnp.int32),
            pltpu.VMEM((VBUF,), jnp.int32),
            pltpu.VMEM((FB,), jnp.int32),
            pltpu.VMEM((FB,), jnp.int32),
            pltpu.VMEM((FB, 128), jnp.float32),
            pltpu.VMEM(((rpt + 1) * 128,), jnp.float32),
            pltpu.SemaphoreType.DMA,
            pltpu.SemaphoreType.DMA,
            pltpu.SemaphoreType.DMA,
        ],
    )(q_pack, h2)

    return out1d.reshape(npad, h)[:n]
